# baseline (speedup 1.0000x reference)
import jax
import jax.numpy as jnp
from jax import lax
from jax.experimental import pallas as pl
from jax.experimental.pallas import tpu as pltpu
from jax.experimental.pallas import tpu_sc as plsc

N_VOCAB = 100000
EMBED_DIM = 64
OFFSET = 20
BATCH = 4096

_info = plsc.get_sparse_core_info()
NC, NS, L = _info.num_cores, _info.num_subcores, _info.num_lanes
NW = NC * NS
DIMS_PER_W = EMBED_DIM // NW
CHUNK_BAGS = 512
CHUNK_IDX = CHUNK_BAGS * OFFSET
N_CHUNKS = BATCH // CHUNK_BAGS
GROUPS = CHUNK_BAGS // L


def _bag_kernel(idx_hbm, wt_hbm, out_hbm, slab_v, idxa_v, idxb_v, acc_v, sem):
    wid = lax.axis_index("s") * NC + lax.axis_index("c")

    for d in range(DIMS_PER_W):
        c = wid * DIMS_PER_W + d
        pltpu.sync_copy(wt_hbm.at[c], slab_v)

        bufs = (idxa_v, idxb_v)
        copies = [
            pltpu.async_copy(
                idx_hbm.at[pl.ds(0 * CHUNK_IDX, CHUNK_IDX)], bufs[0], sem
            )
        ]
        for ch in range(N_CHUNKS):
            copies[ch].wait()
            if ch + 1 < N_CHUNKS:
                copies.append(
                    pltpu.async_copy(
                        idx_hbm.at[pl.ds((ch + 1) * CHUNK_IDX, CHUNK_IDX)],
                        bufs[(ch + 1) % 2],
                        sem,
                    )
                )
            idx_v = bufs[ch % 2]

            @plsc.parallel_loop(0, GROUPS, step=1, unroll=4)
            def group_body(g):
                base = g * L
                parts = []
                for k in range(4):
                    acc = None
                    for r in range(5 * k, 5 * k + 5):
                        iv = idx_v[pl.ds(r * CHUNK_BAGS + base, L)]
                        sv = plsc.load_gather(slab_v, [iv])
                        acc = sv if acc is None else acc + sv
                    parts.append(acc)
                acc_v[pl.ds(ch * CHUNK_BAGS + base, L)] = (
                    (parts[0] + parts[1]) + (parts[2] + parts[3])
                )

        pltpu.sync_copy(acc_v, out_hbm.at[c])


@jax.jit
def _bag(idx, wt):
    mesh = plsc.VectorSubcoreMesh(core_axis_name="c", subcore_axis_name="s")
    return pl.kernel(
        _bag_kernel,
        mesh=mesh,
        compiler_params=pltpu.CompilerParams(needs_layout_passes=False),
        out_type=jax.ShapeDtypeStruct((EMBED_DIM, BATCH), jnp.float32),
        scratch_types=[
            pltpu.VMEM((N_VOCAB,), jnp.float32),
            pltpu.VMEM((CHUNK_IDX,), jnp.int32),
            pltpu.VMEM((CHUNK_IDX,), jnp.int32),
            pltpu.VMEM((BATCH,), jnp.float32),
            pltpu.SemaphoreType.DMA,
        ],
    )(idx, wt)


def kernel(sparse_index_group_batch, sparse_offset_group_batch, weight):
    del sparse_offset_group_batch
    idx = sparse_index_group_batch.astype(jnp.int32)
    idx_t = (
        idx.reshape(N_CHUNKS, CHUNK_BAGS, OFFSET)
        .transpose(0, 2, 1)
        .reshape(-1)
    )
    out_t = _bag(idx_t, weight.T)
    return out_t.T


# speedup vs baseline: 1.2368x; 1.2368x over previous
import jax
import jax.numpy as jnp
from jax import lax
from jax.experimental import pallas as pl
from jax.experimental.pallas import tpu as pltpu
from jax.experimental.pallas import tpu_sc as plsc

N_VOCAB = 100000
EMBED_DIM = 64
OFFSET = 20
BATCH = 4096

_info = plsc.get_sparse_core_info()
NC, NS, L = _info.num_cores, _info.num_subcores, _info.num_lanes
NW = NC * NS
DIMS_PER_W = EMBED_DIM // NW
CHUNK_BAGS = 512
CHUNK_IDX = CHUNK_BAGS * OFFSET
N_CHUNKS = BATCH // CHUNK_BAGS
GROUPS = CHUNK_BAGS // L


def _bag_kernel(idx_hbm, wt_hbm, out_hbm, slab_v, idxa_v, idxb_v, acc_v, sidx_s, sem):
    sid = lax.axis_index("s")
    wid = sid * NC + lax.axis_index("c")

    slab_cp = pltpu.async_copy(wt_hbm.at[wid * DIMS_PER_W], slab_v, sem)

    @pl.when(sid == 0)
    def _stage_idx():
        pltpu.sync_copy(idx_hbm, sidx_s)

    plsc.subcore_barrier()

    for d in range(DIMS_PER_W):
        c = wid * DIMS_PER_W + d
        if d == 0:
            slab_cp.wait()
        else:
            pltpu.sync_copy(wt_hbm.at[c], slab_v)

        bufs = (idxa_v, idxb_v)
        copies = [
            pltpu.async_copy(
                sidx_s.at[pl.ds(0 * CHUNK_IDX, CHUNK_IDX)], bufs[0], sem
            )
        ]
        for ch in range(N_CHUNKS):
            copies[ch].wait()
            if ch + 1 < N_CHUNKS:
                copies.append(
                    pltpu.async_copy(
                        sidx_s.at[pl.ds((ch + 1) * CHUNK_IDX, CHUNK_IDX)],
                        bufs[(ch + 1) % 2],
                        sem,
                    )
                )
            idx_v = bufs[ch % 2]

            @plsc.parallel_loop(0, GROUPS, step=1, unroll=2)
            def group_body(g):
                base = g * L
                parts = []
                for k in range(4):
                    acc = None
                    for r in range(5 * k, 5 * k + 5):
                        iv = idx_v[pl.ds(r * CHUNK_BAGS + base, L)]
                        sv = plsc.load_gather(slab_v, [iv])
                        acc = sv if acc is None else acc + sv
                    parts.append(acc)
                acc_v[pl.ds(ch * CHUNK_BAGS + base, L)] = (
                    (parts[0] + parts[1]) + (parts[2] + parts[3])
                )

        pltpu.sync_copy(acc_v, out_hbm.at[c])


@jax.jit
def _bag(idx, wt):
    mesh = plsc.VectorSubcoreMesh(core_axis_name="c", subcore_axis_name="s")
    return pl.kernel(
        _bag_kernel,
        mesh=mesh,
        compiler_params=pltpu.CompilerParams(needs_layout_passes=False),
        out_type=jax.ShapeDtypeStruct((EMBED_DIM, BATCH), jnp.float32),
        scratch_types=[
            pltpu.VMEM((N_VOCAB,), jnp.float32),
            pltpu.VMEM((CHUNK_IDX,), jnp.int32),
            pltpu.VMEM((CHUNK_IDX,), jnp.int32),
            pltpu.VMEM((BATCH,), jnp.float32),
            pltpu.VMEM_SHARED((BATCH * OFFSET,), jnp.int32),
            pltpu.SemaphoreType.DMA,
        ],
    )(idx, wt)


def kernel(sparse_index_group_batch, sparse_offset_group_batch, weight):
    del sparse_offset_group_batch
    idx = sparse_index_group_batch.astype(jnp.int32)
    idx_t = (
        idx.reshape(N_CHUNKS, CHUNK_BAGS, OFFSET)
        .transpose(0, 2, 1)
        .reshape(-1)
    )
    out_t = _bag(idx_t, weight.T)
    return out_t.T
